# baseline (speedup 1.0000x reference)
import functools

import jax
import jax.numpy as jnp
from jax import lax
from jax.experimental import pallas as pl
from jax.experimental.pallas import tpu as pltpu
from jax.experimental.pallas import tpu_sc as plsc

DIM = 128
CHUNK = 128
NC = 2
NS = 16
NW = NC * NS


def _make_gather(n: int, n_chunks: int, TYPE_ROWS: int):
    cpw = n_chunks // NW
    mesh = plsc.VectorSubcoreMesh(core_axis_name="c", subcore_axis_name="s")

    nbuf = 7
    ahead = 5

    @functools.partial(
        pl.kernel,
        mesh=mesh,
        out_type=jax.ShapeDtypeStruct((n, DIM), jnp.float32),
        scratch_types=[
            pltpu.VMEM((cpw, CHUNK), jnp.int32),
            pltpu.VMEM((nbuf, CHUNK, DIM), jnp.float32),
            pltpu.VMEM_SHARED((TYPE_ROWS, DIM), jnp.float32),
            pltpu.SemaphoreType.DMA((nbuf,)),
            pltpu.SemaphoreType.DMA((nbuf,)),
            pltpu.SemaphoreType.DMA,
        ],
    )
    def gather_kernel(idx_hbm, table_hbm, out_hbm, idx_v, rows_v, table_v,
                      gsem, osem, isem):
        wid = lax.axis_index("s") * NC + lax.axis_index("c")
        cbase = wid * cpw
        idx_copy = pltpu.make_async_copy(idx_hbm.at[wid], idx_v, isem)
        idx_copy.start()

        @pl.when(lax.axis_index("s") == 0)
        def _():
            pltpu.sync_copy(table_hbm, table_v)

        plsc.subcore_barrier()
        idx_copy.wait()

        def ostart(j):
            return lax.min((cbase + j) * CHUNK, n - CHUNK)

        def start_gather(j, b):
            pltpu.async_copy(table_v.at[idx_v.at[j]], rows_v.at[b], gsem.at[b])

        def wait_gather(b):
            pltpu.make_async_copy(
                table_v.at[idx_v.at[0]], rows_v.at[b], gsem.at[b]
            ).wait()

        def start_out(j, b):
            pltpu.async_copy(
                rows_v.at[b], out_hbm.at[pl.ds(ostart(j), CHUNK)], osem.at[b]
            )

        def wait_out(b):
            pltpu.make_async_copy(
                rows_v.at[b], out_hbm.at[pl.ds(0, CHUNK)], osem.at[b]
            ).wait()

        for p in range(min(ahead, cpw)):
            start_gather(p, p)

        def step(j, carry):
            b = lax.rem(j, nbuf)
            wait_gather(b)
            start_out(j, b)

            @pl.when(j + ahead < cpw)
            def _():
                b2 = lax.rem(j + ahead, nbuf)

                @pl.when(j - (nbuf - ahead) >= 0)
                def _():
                    wait_out(b2)

                start_gather(j + ahead, b2)

            return carry

        lax.fori_loop(0, cpw, step, 0)
        for t in range(min(nbuf, cpw)):
            wait_out((cpw - 1 - t) % nbuf)

    return gather_kernel


def kernel(atom_types, table):
    n = atom_types.shape[0]
    n_full = n // CHUNK
    n_chunks = -(-n // CHUNK)
    n_chunks_pad = -(-n_chunks // NW) * NW
    idx_full = atom_types[: n_full * CHUNK].reshape(n_full, CHUNK)
    n_tail = n_chunks_pad - n_full
    idx_tail = jnp.broadcast_to(atom_types[n - CHUNK:], (n_tail, CHUNK))
    idx = jnp.concatenate([idx_full, idx_tail]).reshape(
        NW, n_chunks_pad // NW, CHUNK
    )
    return _make_gather(n, n_chunks_pad, table.shape[0])(idx, table)


# speedup vs baseline: 1.0127x; 1.0127x over previous
import functools

import jax
import jax.numpy as jnp
from jax import lax
from jax.experimental import pallas as pl
from jax.experimental.pallas import tpu as pltpu
from jax.experimental.pallas import tpu_sc as plsc

DIM = 128
CHUNK = 64
NC = 2
NS = 16
NW = NC * NS


def _make_gather(n: int, n_chunks: int, TYPE_ROWS: int):
    cpw = n_chunks // NW
    mesh = plsc.VectorSubcoreMesh(core_axis_name="c", subcore_axis_name="s")

    nbuf = 7
    ahead = 5

    @functools.partial(
        pl.kernel,
        mesh=mesh,
        out_type=jax.ShapeDtypeStruct((n, DIM), jnp.float32),
        scratch_types=[
            pltpu.VMEM((cpw, CHUNK), jnp.int32),
            pltpu.VMEM((nbuf, CHUNK, DIM), jnp.float32),
            pltpu.VMEM_SHARED((TYPE_ROWS, DIM), jnp.float32),
            pltpu.SemaphoreType.DMA((nbuf,)),
            pltpu.SemaphoreType.DMA((nbuf,)),
            pltpu.SemaphoreType.DMA,
        ],
    )
    def gather_kernel(idx_hbm, table_hbm, out_hbm, idx_v, rows_v, table_v,
                      gsem, osem, isem):
        wid = lax.axis_index("s") * NC + lax.axis_index("c")
        cbase = wid * cpw
        idx_copy = pltpu.make_async_copy(idx_hbm.at[wid], idx_v, isem)
        idx_copy.start()

        @pl.when(lax.axis_index("s") == 0)
        def _():
            pltpu.sync_copy(table_hbm, table_v)

        plsc.subcore_barrier()
        idx_copy.wait()

        def ostart(j):
            return lax.min((cbase + j) * CHUNK, n - CHUNK)

        def start_gather(j, b):
            pltpu.async_copy(table_v.at[idx_v.at[j]], rows_v.at[b], gsem.at[b])

        def wait_gather(b):
            pltpu.make_async_copy(
                table_v.at[idx_v.at[0]], rows_v.at[b], gsem.at[b]
            ).wait()

        def start_out(j, b):
            pltpu.async_copy(
                rows_v.at[b], out_hbm.at[pl.ds(ostart(j), CHUNK)], osem.at[b]
            )

        def wait_out(b):
            pltpu.make_async_copy(
                rows_v.at[b], out_hbm.at[pl.ds(0, CHUNK)], osem.at[b]
            ).wait()

        for p in range(min(ahead, cpw)):
            start_gather(p, p)

        def step(j, carry):
            b = lax.rem(j, nbuf)
            wait_gather(b)
            start_out(j, b)

            @pl.when(j + ahead < cpw)
            def _():
                b2 = lax.rem(j + ahead, nbuf)

                @pl.when(j - (nbuf - ahead) >= 0)
                def _():
                    wait_out(b2)

                start_gather(j + ahead, b2)

            return carry

        lax.fori_loop(0, cpw, step, 0)
        for t in range(min(nbuf, cpw)):
            wait_out((cpw - 1 - t) % nbuf)

    return gather_kernel


def kernel(atom_types, table):
    n = atom_types.shape[0]
    n_full = n // CHUNK
    n_chunks = -(-n // CHUNK)
    n_chunks_pad = -(-n_chunks // NW) * NW
    idx_full = atom_types[: n_full * CHUNK].reshape(n_full, CHUNK)
    n_tail = n_chunks_pad - n_full
    idx_tail = jnp.broadcast_to(atom_types[n - CHUNK:], (n_tail, CHUNK))
    idx = jnp.concatenate([idx_full, idx_tail]).reshape(
        NW, n_chunks_pad // NW, CHUNK
    )
    return _make_gather(n, n_chunks_pad, table.shape[0])(idx, table)
